# Initial kernel scaffold; baseline (speedup 1.0000x reference)
import jax
import jax.numpy as jnp
from jax.experimental import pallas as pl


def kernel(user, memory, user_table, item_table, conv_w, conv_b, lin_w, lin_b):
    raise NotImplementedError("write your pallas kernel here")


# speedup vs baseline: 1.5552x; 1.5552x over previous
import functools

import jax
import jax.numpy as jnp
from jax import lax
from jax.experimental import pallas as pl
from jax.experimental.pallas import tpu as pltpu
from jax.experimental.pallas import tpu_sc as plsc

B = 16384
M = 50
D = 64
NC = 2
NS = 16
NW = NC * NS
BPW = B // NW
CB = 8
NCHUNK = BPW // CB
LANES = 16


def _sc_gather_reduce(user, memory, user_table, item_table, wrow):
    mesh = plsc.VectorSubcoreMesh(
        core_axis_name="c", subcore_axis_name="s",
        num_cores=NC, num_subcores=NS,
    )

    @functools.partial(
        pl.kernel,
        mesh=mesh,
        compiler_params=pltpu.CompilerParams(use_tc_tiling_on_sc=False),
        out_type=(
            jax.ShapeDtypeStruct((B, D), jnp.float32),
            jax.ShapeDtypeStruct((B, D), jnp.float32),
        ),
        scratch_types=[
            pltpu.VMEM((CB, M), jnp.int32),
            pltpu.VMEM((CB,), jnp.int32),
            pltpu.VMEM((CB, M, D), jnp.float32),
            pltpu.VMEM((CB, D), jnp.float32),
            pltpu.VMEM((CB, D), jnp.float32),
            pltpu.VMEM((M, D), jnp.float32),
            pltpu.SemaphoreType.DMA,
        ],
    )
    def k(user_hbm, mem_hbm, ut_hbm, it_hbm, w_hbm,
          uemb_hbm, drr_hbm,
          idx_v, uidx_v, rows_v, u_v, drr_v, w_v, sem):
        wid = lax.axis_index("s") * NC + lax.axis_index("c")
        base = wid * BPW
        pltpu.sync_copy(w_hbm, w_v)

        @pl.loop(0, NCHUNK)
        def _(ci):
            b0 = base + ci * CB
            pltpu.sync_copy(mem_hbm.at[pl.ds(b0, CB), :], idx_v)
            pltpu.sync_copy(user_hbm.at[pl.ds(b0, CB)], uidx_v)
            cps = [
                pltpu.async_copy(it_hbm.at[idx_v.at[j]], rows_v.at[j], sem)
                for j in range(CB)
            ]
            ucp = pltpu.async_copy(ut_hbm.at[uidx_v], u_v, sem)
            for c in cps:
                c.wait()
            ucp.wait()

            for v in range(D // LANES):
                sl = pl.ds(v * LANES, LANES)

                def body_m(m, accs, sl=sl):
                    wv = w_v[m, sl]
                    return tuple(
                        accs[j] + rows_v[j, m, sl] * wv for j in range(CB)
                    )

                accs = lax.fori_loop(
                    0, M, body_m,
                    tuple(jnp.zeros((LANES,), jnp.float32) for _ in range(CB)),
                )
                for j in range(CB):
                    drr_v[j, sl] = accs[j]

            pltpu.sync_copy(u_v, uemb_hbm.at[pl.ds(b0, CB), :])
            pltpu.sync_copy(drr_v, drr_hbm.at[pl.ds(b0, CB), :])

    return k(user, memory, user_table, item_table, wrow)


def _tc_combine(u, drr, wt, bias):
    def body(u_ref, d_ref, w_ref, b_ref, o_ref):
        uu = u_ref[...]
        dd = d_ref[...]
        w = w_ref[...]
        acc = jnp.dot(uu, w[:D], preferred_element_type=jnp.float32)
        acc = acc + jnp.dot(uu * dd, w[D:2 * D], preferred_element_type=jnp.float32)
        acc = acc + jnp.dot(dd, w[2 * D:], preferred_element_type=jnp.float32)
        o_ref[...] = acc + b_ref[...]

    return pl.pallas_call(
        body,
        grid=(1,),
        in_specs=[
            pl.BlockSpec((B, D), lambda i: (0, 0)),
            pl.BlockSpec((B, D), lambda i: (0, 0)),
            pl.BlockSpec((3 * D, D), lambda i: (0, 0)),
            pl.BlockSpec((1, D), lambda i: (0, 0)),
        ],
        out_specs=pl.BlockSpec((B, D), lambda i: (0, 0)),
        out_shape=jax.ShapeDtypeStruct((B, D), jnp.float32),
    )(u, drr, wt, bias)


def kernel(user, memory, user_table, item_table, conv_w, conv_b, lin_w, lin_b):
    wrow = jnp.broadcast_to(conv_w[:, None], (M, D))
    wt = lin_w.T
    cb = conv_b[0]
    w1t = wt[:D] + cb * wt[D:2 * D]
    bias = (lin_b + cb * jnp.sum(wt[2 * D:], axis=0)).reshape(1, D)
    wt_folded = jnp.concatenate([w1t, wt[D:2 * D], wt[2 * D:]], axis=0)

    u_emb, drr = _sc_gather_reduce(user, memory, user_table, item_table, wrow)
    return _tc_combine(u_emb, drr, wt_folded, bias)


# speedup vs baseline: 1.7092x; 1.0990x over previous
import functools

import jax
import jax.numpy as jnp
from jax import lax
from jax.experimental import pallas as pl
from jax.experimental.pallas import tpu as pltpu
from jax.experimental.pallas import tpu_sc as plsc

B = 16384
M = 50
D = 64
NC = 2
NS = 16
NW = NC * NS
BPW = B // NW
CB = 8
NCHUNK = BPW // CB
LANES = 16


def _sc_gather_reduce(user, memory, user_table, item_table, wrow):
    mesh = plsc.VectorSubcoreMesh(
        core_axis_name="c", subcore_axis_name="s",
        num_cores=NC, num_subcores=NS,
    )

    @functools.partial(
        pl.kernel,
        mesh=mesh,
        compiler_params=pltpu.CompilerParams(use_tc_tiling_on_sc=False),
        out_type=(
            jax.ShapeDtypeStruct((B, D), jnp.float32),
            jax.ShapeDtypeStruct((B, D), jnp.float32),
        ),
        scratch_types=[
            pltpu.VMEM((2, CB, M), jnp.int32),
            pltpu.VMEM((2, CB), jnp.int32),
            pltpu.VMEM((2, CB, M, D), jnp.float32),
            pltpu.VMEM((2, CB, D), jnp.float32),
            pltpu.VMEM((2, CB, D), jnp.float32),
            pltpu.VMEM((M, D), jnp.float32),
            pltpu.SemaphoreType.DMA,
            pltpu.SemaphoreType.DMA,
            pltpu.SemaphoreType.DMA,
            pltpu.SemaphoreType.DMA,
            pltpu.SemaphoreType.DMA,
            pltpu.SemaphoreType.DMA,
        ],
    )
    def k(user_hbm, mem_hbm, ut_hbm, it_hbm, w_hbm,
          uemb_hbm, drr_hbm,
          idx_v, uidx_v, rows_v, u_v, drr_v, w_v,
          isem0, isem1, rsem0, rsem1, wsem0, wsem1):
        isem = (isem0, isem1)
        rsem = (rsem0, rsem1)
        wsem = (wsem0, wsem1)
        wid = lax.axis_index("s") * NC + lax.axis_index("c")
        base = wid * BPW
        pltpu.sync_copy(w_hbm, w_v)

        def idx_load(p, ci):
            b0 = base + ci * CB
            pltpu.async_copy(mem_hbm.at[pl.ds(b0, CB), :], idx_v.at[p],
                             isem[p])
            pltpu.async_copy(user_hbm.at[pl.ds(b0, CB)], uidx_v.at[p],
                             isem[p])

        def idx_wait(p):
            pltpu.make_async_copy(mem_hbm.at[pl.ds(0, CB), :], idx_v.at[p],
                                  isem[p]).wait()
            pltpu.make_async_copy(user_hbm.at[pl.ds(0, CB)], uidx_v.at[p],
                                  isem[p]).wait()

        def rows_fire(p):
            for j in range(CB):
                pltpu.async_copy(it_hbm.at[idx_v.at[p].at[j]],
                                 rows_v.at[p].at[j], rsem[p])
            pltpu.async_copy(ut_hbm.at[uidx_v.at[p]], u_v.at[p], rsem[p])

        def rows_wait(p):
            for j in range(CB):
                pltpu.make_async_copy(it_hbm.at[idx_v.at[p].at[j]],
                                      rows_v.at[p].at[j], rsem[p]).wait()
            pltpu.make_async_copy(ut_hbm.at[uidx_v.at[p]], u_v.at[p],
                                  rsem[p]).wait()

        def compute(p):
            for v in range(D // LANES):
                sl = pl.ds(v * LANES, LANES)

                def body_m(m, accs, sl=sl, p=p):
                    wv = w_v[m, sl]
                    return tuple(
                        accs[j] + rows_v[p, j, m, sl] * wv for j in range(CB)
                    )

                accs = lax.fori_loop(
                    0, M, body_m,
                    tuple(jnp.zeros((LANES,), jnp.float32) for _ in range(CB)),
                )
                for j in range(CB):
                    drr_v[p, j, sl] = accs[j]

        def out_write(p, ci):
            b0 = base + ci * CB
            pltpu.async_copy(u_v.at[p], uemb_hbm.at[pl.ds(b0, CB), :],
                             wsem[p])
            pltpu.async_copy(drr_v.at[p], drr_hbm.at[pl.ds(b0, CB), :],
                             wsem[p])

        def out_wait(p):
            pltpu.make_async_copy(u_v.at[p], uemb_hbm.at[pl.ds(0, CB), :],
                                  wsem[p]).wait()
            pltpu.make_async_copy(drr_v.at[p], drr_hbm.at[pl.ds(0, CB), :],
                                  wsem[p]).wait()

        idx_load(0, 0)
        idx_load(1, 1)
        idx_wait(0)
        rows_fire(0)

        def step(ci, p):
            q = 1 - p

            @pl.when(ci + 1 < NCHUNK)
            def _():
                idx_wait(q)
                rows_fire(q)

            rows_wait(p)

            @pl.when(ci >= 2)
            def _():
                out_wait(p)

            compute(p)
            out_write(p, ci)

            @pl.when(ci + 2 < NCHUNK)
            def _():
                idx_load(p, ci + 2)

        @pl.loop(0, NCHUNK // 2)
        def _(kk):
            step(2 * kk, 0)
            step(2 * kk + 1, 1)

        out_wait(0)
        out_wait(1)

    return k(user, memory, user_table, item_table, wrow)


def _tc_combine(u, drr, wt, bias):
    def body(u_ref, d_ref, w_ref, b_ref, o_ref):
        uu = u_ref[...]
        dd = d_ref[...]
        w = w_ref[...]
        acc = jnp.dot(uu, w[:D], preferred_element_type=jnp.float32)
        acc = acc + jnp.dot(uu * dd, w[D:2 * D], preferred_element_type=jnp.float32)
        acc = acc + jnp.dot(dd, w[2 * D:], preferred_element_type=jnp.float32)
        o_ref[...] = acc + b_ref[...]

    return pl.pallas_call(
        body,
        grid=(1,),
        in_specs=[
            pl.BlockSpec((B, D), lambda i: (0, 0)),
            pl.BlockSpec((B, D), lambda i: (0, 0)),
            pl.BlockSpec((3 * D, D), lambda i: (0, 0)),
            pl.BlockSpec((1, D), lambda i: (0, 0)),
        ],
        out_specs=pl.BlockSpec((B, D), lambda i: (0, 0)),
        out_shape=jax.ShapeDtypeStruct((B, D), jnp.float32),
    )(u, drr, wt, bias)


def kernel(user, memory, user_table, item_table, conv_w, conv_b, lin_w, lin_b):
    wrow = jnp.broadcast_to(conv_w[:, None], (M, D))
    wt = lin_w.T
    cb = conv_b[0]
    w1t = wt[:D] + cb * wt[D:2 * D]
    bias = (lin_b + cb * jnp.sum(wt[2 * D:], axis=0)).reshape(1, D)
    wt_folded = jnp.concatenate([w1t, wt[D:2 * D], wt[2 * D:]], axis=0)

    u_emb, drr = _sc_gather_reduce(user, memory, user_table, item_table, wrow)
    return _tc_combine(u_emb, drr, wt_folded, bias)


# speedup vs baseline: 2.2080x; 1.2918x over previous
import functools

import jax
import jax.numpy as jnp
from jax import lax
from jax.experimental import pallas as pl
from jax.experimental.pallas import tpu as pltpu
from jax.experimental.pallas import tpu_sc as plsc

B = 16384
M = 50
D = 64
NC = 2
NS = 16
NW = NC * NS
BPW = B // NW
CB = 8
NCHUNK = BPW // CB
LANES = 16
TBLK = 4096


def _tc_dup_transpose(table):
    n = table.shape[0]
    nblk = (n + TBLK - 1) // TBLK
    npad = nblk * TBLK
    tt = jnp.swapaxes(table, 0, 1)

    def body(x_ref, o_ref):
        y = jnp.swapaxes(x_ref[...], 0, 1)
        o_ref[...] = jnp.concatenate([y, y], axis=1)

    out = pl.pallas_call(
        body,
        grid=(nblk,),
        in_specs=[pl.BlockSpec((D, TBLK), lambda i: (0, i))],
        out_specs=pl.BlockSpec((TBLK, 2 * D), lambda i: (i, 0)),
        out_shape=jax.ShapeDtypeStruct((npad, 2 * D), jnp.float32),
    )(tt)
    return out.reshape(2 * npad, D)


def _sc_gather_reduce(user, memory, user_table, item_table, wrow):
    mesh = plsc.VectorSubcoreMesh(
        core_axis_name="c", subcore_axis_name="s",
        num_cores=NC, num_subcores=NS,
    )

    @functools.partial(
        pl.kernel,
        mesh=mesh,
        compiler_params=pltpu.CompilerParams(use_tc_tiling_on_sc=False),
        out_type=(
            jax.ShapeDtypeStruct((B, D), jnp.float32),
            jax.ShapeDtypeStruct((B, D), jnp.float32),
        ),
        scratch_types=[
            pltpu.VMEM((2, CB, M), jnp.int32),
            pltpu.VMEM((2, CB), jnp.int32),
            pltpu.VMEM((2, CB, M, D), jnp.float32),
            pltpu.VMEM((2, CB, D), jnp.float32),
            pltpu.VMEM((2, CB, D), jnp.float32),
            pltpu.VMEM((M, D), jnp.float32),
            pltpu.SemaphoreType.DMA,
            pltpu.SemaphoreType.DMA,
            pltpu.SemaphoreType.DMA,
            pltpu.SemaphoreType.DMA,
            pltpu.SemaphoreType.DMA,
            pltpu.SemaphoreType.DMA,
        ],
    )
    def k(user_hbm, mem_hbm, ut_hbm, it_hbm, w_hbm,
          uemb_hbm, drr_hbm,
          idx_v, uidx_v, rows_v, u_v, drr_v, w_v,
          isem0, isem1, rsem0, rsem1, wsem0, wsem1):
        isem = (isem0, isem1)
        rsem = (rsem0, rsem1)
        wsem = (wsem0, wsem1)
        wid = lax.axis_index("s") * NC + lax.axis_index("c")
        base = wid * BPW
        pltpu.sync_copy(w_hbm, w_v)

        def idx_load(p, ci):
            b0 = base + ci * CB
            pltpu.async_copy(mem_hbm.at[pl.ds(b0, CB), :], idx_v.at[p],
                             isem[p])
            pltpu.async_copy(user_hbm.at[pl.ds(b0, CB)], uidx_v.at[p],
                             isem[p])

        def idx_wait(p):
            pltpu.make_async_copy(mem_hbm.at[pl.ds(0, CB), :], idx_v.at[p],
                                  isem[p]).wait()
            pltpu.make_async_copy(user_hbm.at[pl.ds(0, CB)], uidx_v.at[p],
                                  isem[p]).wait()

        def rows_fire(p):
            for j in range(CB):
                pltpu.async_copy(it_hbm.at[idx_v.at[p].at[j]],
                                 rows_v.at[p].at[j], rsem[p])
            pltpu.async_copy(ut_hbm.at[uidx_v.at[p]], u_v.at[p], rsem[p])

        def rows_wait(p):
            for j in range(CB):
                pltpu.make_async_copy(it_hbm.at[idx_v.at[p].at[j]],
                                      rows_v.at[p].at[j], rsem[p]).wait()
            pltpu.make_async_copy(ut_hbm.at[uidx_v.at[p]], u_v.at[p],
                                  rsem[p]).wait()

        def compute(p):
            for v in range(D // LANES):
                sl = pl.ds(v * LANES, LANES)

                def body_m(m, accs, sl=sl, p=p):
                    wv = w_v[m, sl]
                    return tuple(
                        accs[j] + rows_v[p, j, m, sl] * wv for j in range(CB)
                    )

                accs = lax.fori_loop(
                    0, M, body_m,
                    tuple(jnp.zeros((LANES,), jnp.float32) for _ in range(CB)),
                )
                for j in range(CB):
                    drr_v[p, j, sl] = accs[j]

        def out_write(p, ci):
            b0 = base + ci * CB
            pltpu.async_copy(u_v.at[p], uemb_hbm.at[pl.ds(b0, CB), :],
                             wsem[p])
            pltpu.async_copy(drr_v.at[p], drr_hbm.at[pl.ds(b0, CB), :],
                             wsem[p])

        def out_wait(p):
            pltpu.make_async_copy(u_v.at[p], uemb_hbm.at[pl.ds(0, CB), :],
                                  wsem[p]).wait()
            pltpu.make_async_copy(drr_v.at[p], drr_hbm.at[pl.ds(0, CB), :],
                                  wsem[p]).wait()

        idx_load(0, 0)
        idx_load(1, 1)
        idx_wait(0)
        rows_fire(0)

        def step(ci, p):
            q = 1 - p

            @pl.when(ci + 1 < NCHUNK)
            def _():
                idx_wait(q)
                rows_fire(q)

            rows_wait(p)

            @pl.when(ci >= 2)
            def _():
                out_wait(p)

            compute(p)
            out_write(p, ci)

            @pl.when(ci + 2 < NCHUNK)
            def _():
                idx_load(p, ci + 2)

        @pl.loop(0, NCHUNK // 2)
        def _(kk):
            step(2 * kk, 0)
            step(2 * kk + 1, 1)

        out_wait(0)
        out_wait(1)

    return k(user, memory, user_table, item_table, wrow)


def _tc_combine(u, drr, wt, bias):
    def body(u_ref, d_ref, w_ref, b_ref, o_ref):
        uu = u_ref[...]
        dd = d_ref[...]
        w = w_ref[...]
        acc = jnp.dot(uu, w[:D], preferred_element_type=jnp.float32)
        acc = acc + jnp.dot(uu * dd, w[D:2 * D], preferred_element_type=jnp.float32)
        acc = acc + jnp.dot(dd, w[2 * D:], preferred_element_type=jnp.float32)
        o_ref[...] = acc + b_ref[...]

    return pl.pallas_call(
        body,
        grid=(1,),
        in_specs=[
            pl.BlockSpec((B, D), lambda i: (0, 0)),
            pl.BlockSpec((B, D), lambda i: (0, 0)),
            pl.BlockSpec((3 * D, D), lambda i: (0, 0)),
            pl.BlockSpec((1, D), lambda i: (0, 0)),
        ],
        out_specs=pl.BlockSpec((B, D), lambda i: (0, 0)),
        out_shape=jax.ShapeDtypeStruct((B, D), jnp.float32),
    )(u, drr, wt, bias)


def kernel(user, memory, user_table, item_table, conv_w, conv_b, lin_w, lin_b):
    wrow = jnp.broadcast_to(conv_w[:, None], (M, D))
    wt = lin_w.T
    cb = conv_b[0]
    w1t = wt[:D] + cb * wt[D:2 * D]
    bias = (lin_b + cb * jnp.sum(wt[2 * D:], axis=0)).reshape(1, D)
    wt_folded = jnp.concatenate([w1t, wt[D:2 * D], wt[2 * D:]], axis=0)

    ut2 = _tc_dup_transpose(user_table)
    it2 = _tc_dup_transpose(item_table)
    u_emb, drr = _sc_gather_reduce(user * 2, memory * 2, ut2, it2, wrow)
    return _tc_combine(u_emb, drr, wt_folded, bias)


# speedup vs baseline: 2.5900x; 1.1730x over previous
import functools

import jax
import jax.numpy as jnp
from jax import lax
from jax.experimental import pallas as pl
from jax.experimental.pallas import tpu as pltpu
from jax.experimental.pallas import tpu_sc as plsc

B = 16384
M = 50
D = 64
NC = 2
NS = 16
NW = NC * NS
BPW = B // NW
CB = 8
NCHUNK = BPW // CB
LANES = 16
TBLK = 8192


def _tc_dup_transpose(table):
    n = table.shape[0]
    nblk = (n + TBLK - 1) // TBLK
    npad = nblk * TBLK
    tt = jnp.swapaxes(table, 0, 1)

    def body(x_ref, o_ref):
        y = jnp.swapaxes(x_ref[...], 0, 1)
        o_ref[:, 0:D] = y
        o_ref[:, D:2 * D] = y

    out = pl.pallas_call(
        body,
        grid=(nblk,),
        in_specs=[pl.BlockSpec((D, TBLK), lambda i: (0, i))],
        out_specs=pl.BlockSpec((TBLK, 2 * D), lambda i: (i, 0)),
        out_shape=jax.ShapeDtypeStruct((npad, 2 * D), jnp.float32),
    )(tt)
    return out.reshape(2 * npad, D)


def _sc_item_reduce(memory, item_table, wrow):
    mesh = plsc.VectorSubcoreMesh(
        core_axis_name="c", subcore_axis_name="s",
        num_cores=NC, num_subcores=NS,
    )

    @functools.partial(
        pl.kernel,
        mesh=mesh,
        compiler_params=pltpu.CompilerParams(use_tc_tiling_on_sc=False),
        out_type=jax.ShapeDtypeStruct((B, D), jnp.float32),
        scratch_types=[
            pltpu.VMEM((2, CB, M), jnp.int32),
            pltpu.VMEM((2, CB, M, D), jnp.float32),
            pltpu.VMEM((2, CB, D), jnp.float32),
            pltpu.VMEM((M, D), jnp.float32),
            pltpu.SemaphoreType.DMA,
            pltpu.SemaphoreType.DMA,
            pltpu.SemaphoreType.DMA,
            pltpu.SemaphoreType.DMA,
            pltpu.SemaphoreType.DMA,
            pltpu.SemaphoreType.DMA,
        ],
    )
    def k(mem_hbm, it_hbm, w_hbm, drr_hbm,
          idx_v, rows_v, drr_v, w_v,
          isem0, isem1, rsem0, rsem1, wsem0, wsem1):
        isem = (isem0, isem1)
        rsem = (rsem0, rsem1)
        wsem = (wsem0, wsem1)
        wid = lax.axis_index("s") * NC + lax.axis_index("c")
        base = wid * BPW
        pltpu.sync_copy(w_hbm, w_v)

        def idx_load(p, ci):
            b0 = base + ci * CB
            pltpu.async_copy(mem_hbm.at[pl.ds(b0, CB), :], idx_v.at[p],
                             isem[p])

        def idx_wait(p):
            pltpu.make_async_copy(mem_hbm.at[pl.ds(0, CB), :], idx_v.at[p],
                                  isem[p]).wait()

        def rows_fire(p):
            for j in range(CB):
                pltpu.async_copy(it_hbm.at[idx_v.at[p].at[j]],
                                 rows_v.at[p].at[j], rsem[p])

        def rows_wait(p):
            for j in range(CB):
                pltpu.make_async_copy(it_hbm.at[idx_v.at[p].at[j]],
                                      rows_v.at[p].at[j], rsem[p]).wait()

        def compute(p):
            for v in range(D // LANES):
                sl = pl.ds(v * LANES, LANES)

                def body_m(m, accs, sl=sl, p=p):
                    wv = w_v[m, sl]
                    return tuple(
                        accs[j] + rows_v[p, j, m, sl] * wv for j in range(CB)
                    )

                accs = lax.fori_loop(
                    0, M, body_m,
                    tuple(jnp.zeros((LANES,), jnp.float32) for _ in range(CB)),
                )
                for j in range(CB):
                    drr_v[p, j, sl] = accs[j]

        def out_write(p, ci):
            b0 = base + ci * CB
            pltpu.async_copy(drr_v.at[p], drr_hbm.at[pl.ds(b0, CB), :],
                             wsem[p])

        def out_wait(p):
            pltpu.make_async_copy(drr_v.at[p], drr_hbm.at[pl.ds(0, CB), :],
                                  wsem[p]).wait()

        idx_load(0, 0)
        idx_load(1, 1)
        idx_wait(0)
        rows_fire(0)

        def step(ci, p):
            q = 1 - p

            @pl.when(ci + 1 < NCHUNK)
            def _():
                idx_wait(q)
                rows_fire(q)

            rows_wait(p)

            @pl.when(ci >= 2)
            def _():
                out_wait(p)

            compute(p)
            out_write(p, ci)

            @pl.when(ci + 2 < NCHUNK)
            def _():
                idx_load(p, ci + 2)

        @pl.loop(0, NCHUNK // 2)
        def _(kk):
            step(2 * kk, 0)
            step(2 * kk + 1, 1)

        out_wait(0)
        out_wait(1)

    return k(memory, item_table, wrow)


def _sc_user_gather(user, user_table):
    mesh = plsc.VectorSubcoreMesh(
        core_axis_name="c", subcore_axis_name="s",
        num_cores=NC, num_subcores=NS,
    )

    @functools.partial(
        pl.kernel,
        mesh=mesh,
        compiler_params=pltpu.CompilerParams(use_tc_tiling_on_sc=False),
        out_type=jax.ShapeDtypeStruct((B, D), jnp.float32),
        scratch_types=[
            pltpu.VMEM((BPW,), jnp.int32),
            pltpu.VMEM((BPW, D), jnp.float32),
            pltpu.SemaphoreType.DMA,
        ],
    )
    def k(user_hbm, ut_hbm, uemb_hbm, uidx_v, u_v, sem):
        wid = lax.axis_index("s") * NC + lax.axis_index("c")
        base = wid * BPW
        pltpu.sync_copy(user_hbm.at[pl.ds(base, BPW)], uidx_v)
        for t in range(BPW // 128):
            pltpu.async_copy(ut_hbm.at[uidx_v.at[pl.ds(t * 128, 128)]],
                             u_v.at[pl.ds(t * 128, 128)], sem)
        for t in range(BPW // 128):
            pltpu.make_async_copy(ut_hbm.at[uidx_v.at[pl.ds(t * 128, 128)]],
                                  u_v.at[pl.ds(t * 128, 128)], sem).wait()
        pltpu.sync_copy(u_v, uemb_hbm.at[pl.ds(base, BPW), :])

    return k(user, user_table)


def _tc_combine(u, drr, wt, bias):
    def body(u_ref, d_ref, w_ref, b_ref, o_ref):
        uu = u_ref[...]
        dd = d_ref[...]
        w = w_ref[...]
        acc = jnp.dot(uu, w[:D], preferred_element_type=jnp.float32)
        acc = acc + jnp.dot(uu * dd, w[D:2 * D], preferred_element_type=jnp.float32)
        acc = acc + jnp.dot(dd, w[2 * D:], preferred_element_type=jnp.float32)
        o_ref[...] = acc + b_ref[...]

    return pl.pallas_call(
        body,
        grid=(1,),
        in_specs=[
            pl.BlockSpec((B, D), lambda i: (0, 0)),
            pl.BlockSpec((B, D), lambda i: (0, 0)),
            pl.BlockSpec((3 * D, D), lambda i: (0, 0)),
            pl.BlockSpec((1, D), lambda i: (0, 0)),
        ],
        out_specs=pl.BlockSpec((B, D), lambda i: (0, 0)),
        out_shape=jax.ShapeDtypeStruct((B, D), jnp.float32),
    )(u, drr, wt, bias)


def kernel(user, memory, user_table, item_table, conv_w, conv_b, lin_w, lin_b):
    wrow = jnp.broadcast_to(conv_w[:, None], (M, D))
    wt = lin_w.T
    cb = conv_b[0]
    w1t = wt[:D] + cb * wt[D:2 * D]
    bias = (lin_b + cb * jnp.sum(wt[2 * D:], axis=0)).reshape(1, D)
    wt_folded = jnp.concatenate([w1t, wt[D:2 * D], wt[2 * D:]], axis=0)

    it2 = _tc_dup_transpose(item_table)
    drr = _sc_item_reduce(memory * 2, it2, wrow)
    ut2 = _tc_dup_transpose(user_table)
    u_emb = _sc_user_gather(user * 2, ut2)
    return _tc_combine(u_emb, drr, wt_folded, bias)


# speedup vs baseline: 2.8167x; 1.0875x over previous
import functools

import jax
import jax.numpy as jnp
from jax import lax
from jax.experimental import pallas as pl
from jax.experimental.pallas import tpu as pltpu
from jax.experimental.pallas import tpu_sc as plsc

B = 16384
M = 50
D = 64
NC = 2
NS = 16
NW = NC * NS
BPW = B // NW
CB = 8
NCHUNK = BPW // CB
LANES = 16
TBLK = 16384


def _tc_dup_transpose(table):
    n = table.shape[0]
    nblk = (n + TBLK - 1) // TBLK
    npad = nblk * TBLK
    tt = jnp.swapaxes(table, 0, 1)

    def body(x_ref, o_ref):
        y = jnp.swapaxes(x_ref[...], 0, 1)
        o_ref[:, 0:D] = y
        o_ref[:, D:2 * D] = y

    out = pl.pallas_call(
        body,
        grid=(nblk,),
        in_specs=[pl.BlockSpec((D, TBLK), lambda i: (0, i))],
        out_specs=pl.BlockSpec((TBLK, 2 * D), lambda i: (i, 0)),
        out_shape=jax.ShapeDtypeStruct((npad, 2 * D), jnp.float32),
    )(tt)
    return out.reshape(2 * npad, D)


def _sc_item_reduce(memory, item_table, wrow):
    mesh = plsc.VectorSubcoreMesh(
        core_axis_name="c", subcore_axis_name="s",
        num_cores=NC, num_subcores=NS,
    )

    @functools.partial(
        pl.kernel,
        mesh=mesh,
        compiler_params=pltpu.CompilerParams(use_tc_tiling_on_sc=False),
        out_type=jax.ShapeDtypeStruct((B, D), jnp.float32),
        scratch_types=[
            pltpu.VMEM((2, CB, M), jnp.int32),
            pltpu.VMEM((2, CB, M, D), jnp.float32),
            pltpu.VMEM((2, CB, D), jnp.float32),
            pltpu.VMEM((M, D), jnp.float32),
            pltpu.SemaphoreType.DMA,
            pltpu.SemaphoreType.DMA,
            pltpu.SemaphoreType.DMA,
            pltpu.SemaphoreType.DMA,
            pltpu.SemaphoreType.DMA,
            pltpu.SemaphoreType.DMA,
        ],
    )
    def k(mem_hbm, it_hbm, w_hbm, drr_hbm,
          idx_v, rows_v, drr_v, w_v,
          isem0, isem1, rsem0, rsem1, wsem0, wsem1):
        isem = (isem0, isem1)
        rsem = (rsem0, rsem1)
        wsem = (wsem0, wsem1)
        wid = lax.axis_index("s") * NC + lax.axis_index("c")
        base = wid * BPW
        pltpu.sync_copy(w_hbm, w_v)

        def idx_load(p, ci):
            b0 = base + ci * CB
            pltpu.async_copy(mem_hbm.at[pl.ds(b0, CB), :], idx_v.at[p],
                             isem[p])

        def idx_wait(p):
            pltpu.make_async_copy(mem_hbm.at[pl.ds(0, CB), :], idx_v.at[p],
                                  isem[p]).wait()

        def rows_fire(p):
            for j in range(CB):
                pltpu.async_copy(it_hbm.at[idx_v.at[p].at[j]],
                                 rows_v.at[p].at[j], rsem[p])

        def rows_wait(p):
            for j in range(CB):
                pltpu.make_async_copy(it_hbm.at[idx_v.at[p].at[j]],
                                      rows_v.at[p].at[j], rsem[p]).wait()

        def compute(p):
            for v in range(D // LANES):
                sl = pl.ds(v * LANES, LANES)

                def body_m(m, accs, sl=sl, p=p):
                    wv = w_v[m, sl]
                    return tuple(
                        accs[j] + rows_v[p, j, m, sl] * wv for j in range(CB)
                    )

                accs = lax.fori_loop(
                    0, M, body_m,
                    tuple(jnp.zeros((LANES,), jnp.float32) for _ in range(CB)),
                )
                for j in range(CB):
                    drr_v[p, j, sl] = accs[j]

        def out_write(p, ci):
            b0 = base + ci * CB
            pltpu.async_copy(drr_v.at[p], drr_hbm.at[pl.ds(b0, CB), :],
                             wsem[p])

        def out_wait(p):
            pltpu.make_async_copy(drr_v.at[p], drr_hbm.at[pl.ds(0, CB), :],
                                  wsem[p]).wait()

        idx_load(0, 0)
        idx_load(1, 1)
        idx_wait(0)
        rows_fire(0)

        def step(ci, p):
            q = 1 - p

            @pl.when(ci + 1 < NCHUNK)
            def _():
                idx_wait(q)
                rows_fire(q)

            rows_wait(p)

            @pl.when(ci >= 2)
            def _():
                out_wait(p)

            compute(p)
            out_write(p, ci)

            @pl.when(ci + 2 < NCHUNK)
            def _():
                idx_load(p, ci + 2)

        @pl.loop(0, NCHUNK // 2)
        def _(kk):
            step(2 * kk, 0)
            step(2 * kk + 1, 1)

        out_wait(0)
        out_wait(1)

    return k(memory, item_table, wrow)


def _sc_user_gather(user, user_table):
    mesh = plsc.VectorSubcoreMesh(
        core_axis_name="c", subcore_axis_name="s",
        num_cores=NC, num_subcores=NS,
    )

    @functools.partial(
        pl.kernel,
        mesh=mesh,
        compiler_params=pltpu.CompilerParams(use_tc_tiling_on_sc=False),
        out_type=jax.ShapeDtypeStruct((B, D), jnp.float32),
        scratch_types=[
            pltpu.VMEM((BPW,), jnp.int32),
            pltpu.VMEM((BPW, D), jnp.float32),
            pltpu.SemaphoreType.DMA,
        ],
    )
    def k(user_hbm, ut_hbm, uemb_hbm, uidx_v, u_v, sem):
        wid = lax.axis_index("s") * NC + lax.axis_index("c")
        base = wid * BPW
        pltpu.sync_copy(user_hbm.at[pl.ds(base, BPW)], uidx_v)
        for t in range(BPW // 128):
            pltpu.async_copy(ut_hbm.at[uidx_v.at[pl.ds(t * 128, 128)]],
                             u_v.at[pl.ds(t * 128, 128)], sem)
        for t in range(BPW // 128):
            pltpu.make_async_copy(ut_hbm.at[uidx_v.at[pl.ds(t * 128, 128)]],
                                  u_v.at[pl.ds(t * 128, 128)], sem).wait()
        pltpu.sync_copy(u_v, uemb_hbm.at[pl.ds(base, BPW), :])

    return k(user, user_table)


def _tc_combine(u, drr, wt, bias):
    def body(u_ref, d_ref, w_ref, b_ref, o_ref):
        uu = u_ref[...]
        dd = d_ref[...]
        w = w_ref[...]
        acc = jnp.dot(uu, w[:D], preferred_element_type=jnp.float32)
        acc = acc + jnp.dot(uu * dd, w[D:2 * D], preferred_element_type=jnp.float32)
        acc = acc + jnp.dot(dd, w[2 * D:], preferred_element_type=jnp.float32)
        o_ref[...] = acc + b_ref[...]

    return pl.pallas_call(
        body,
        grid=(1,),
        in_specs=[
            pl.BlockSpec((B, D), lambda i: (0, 0)),
            pl.BlockSpec((B, D), lambda i: (0, 0)),
            pl.BlockSpec((3 * D, D), lambda i: (0, 0)),
            pl.BlockSpec((1, D), lambda i: (0, 0)),
        ],
        out_specs=pl.BlockSpec((B, D), lambda i: (0, 0)),
        out_shape=jax.ShapeDtypeStruct((B, D), jnp.float32),
    )(u, drr, wt, bias)


def kernel(user, memory, user_table, item_table, conv_w, conv_b, lin_w, lin_b):
    wrow = jnp.broadcast_to(conv_w[:, None], (M, D))
    wt = lin_w.T
    cb = conv_b[0]
    w1t = wt[:D] + cb * wt[D:2 * D]
    bias = (lin_b + cb * jnp.sum(wt[2 * D:], axis=0)).reshape(1, D)
    wt_folded = jnp.concatenate([w1t, wt[D:2 * D], wt[2 * D:]], axis=0)

    it2 = _tc_dup_transpose(item_table)
    drr = _sc_item_reduce(memory * 2, it2, wrow)
    ut2 = _tc_dup_transpose(user_table)
    u_emb = _sc_user_gather(user * 2, ut2)
    return _tc_combine(u_emb, drr, wt_folded, bias)
